# Initial kernel scaffold; baseline (speedup 1.0000x reference)
import jax
import jax.numpy as jnp
from jax.experimental import pallas as pl


def kernel(x, edge_index, batch, W1, a_src1, a_dst1, b1, g1, be1, W2, a_src2, a_dst2, b2, g2, be2, fw1, fb1, fw2, fb2):
    raise NotImplementedError("write your pallas kernel here")


# speedup vs baseline: 1.1743x; 1.1743x over previous
import functools

import jax
import jax.numpy as jnp
from jax.experimental import pallas as pl
from jax.experimental.pallas import tpu as pltpu

_NBLK = 2000


def _dense1_body(x_ref, w_ref, asrc_ref, adst_ref, h_ref, as_ref, ad_ref):
    h = jnp.dot(x_ref[...], w_ref[...], preferred_element_type=jnp.float32)
    h_ref[...] = h
    as_ref[...] = jnp.dot(h, asrc_ref[...], preferred_element_type=jnp.float32)
    ad_ref[...] = jnp.dot(h, adst_ref[...], preferred_element_type=jnp.float32)


def _dense1(x, W1, Asrc, Adst, heads):
    n = x.shape[0]
    d_out = W1.shape[1]
    grid = n // _NBLK
    return pl.pallas_call(
        _dense1_body,
        grid=(grid,),
        in_specs=[
            pl.BlockSpec((_NBLK, x.shape[1]), lambda i: (i, 0)),
            pl.BlockSpec((W1.shape[0], d_out), lambda i: (0, 0)),
            pl.BlockSpec((d_out, heads), lambda i: (0, 0)),
            pl.BlockSpec((d_out, heads), lambda i: (0, 0)),
        ],
        out_specs=[
            pl.BlockSpec((_NBLK, d_out), lambda i: (i, 0)),
            pl.BlockSpec((_NBLK, heads), lambda i: (i, 0)),
            pl.BlockSpec((_NBLK, heads), lambda i: (i, 0)),
        ],
        out_shape=[
            jax.ShapeDtypeStruct((n, d_out), jnp.float32),
            jax.ShapeDtypeStruct((n, heads), jnp.float32),
            jax.ShapeDtypeStruct((n, heads), jnp.float32),
        ],
    )(x, W1, Asrc, Adst)


def _block_diag_a(a):
    heads, c = a.shape
    eye = jnp.eye(heads, dtype=a.dtype)
    return (a[:, :, None] * eye[:, None, :]).reshape(heads * c, heads)


def _gat_jnp(h, alpha_s, alpha_d, src, dst, heads, out_ch):
    n = h.shape[0]
    e = alpha_s[src] + alpha_d[dst]
    e = jax.nn.leaky_relu(e, negative_slope=0.2)
    ex = jnp.exp(e)
    denom = jax.ops.segment_sum(ex, dst, num_segments=n)
    msg = h[src].reshape(-1, heads, out_ch) * ex[:, :, None]
    num = jax.ops.segment_sum(msg, dst, num_segments=n)
    out = num / (denom + 1e-16)[:, :, None]
    return out


def _batch_norm(x, gamma, beta, eps=1e-5):
    mu = jnp.mean(x, axis=0)
    var = jnp.var(x, axis=0)
    return (x - mu) / jnp.sqrt(var + eps) * gamma + beta


def kernel(x, edge_index, batch, W1, a_src1, a_dst1, b1, g1, be1,
           W2, a_src2, a_dst2, b2, g2, be2, fw1, fb1, fw2, fb2):
    n = x.shape[0]
    B = 128
    loops = jnp.arange(n, dtype=edge_index.dtype)
    src = jnp.concatenate([edge_index[0], loops])
    dst = jnp.concatenate([edge_index[1], loops])

    h1, as1, ad1 = _dense1(x, W1, _block_diag_a(a_src1), _block_diag_a(a_dst1), 4)
    o1 = _gat_jnp(h1, as1, ad1, src, dst, 4, 64).reshape(n, 256) + b1
    h1e = jax.nn.elu(_batch_norm(o1, g1, be1))

    h2, as2, ad2 = _dense1(h1e, W2, _block_diag_a(a_src2), _block_diag_a(a_dst2), 1)
    o2 = _gat_jnp(h2, as2, ad2, src, dst, 1, 64).reshape(n, 64) + b2
    h2e = jax.nn.elu(_batch_norm(o2, g2, be2))

    counts = jax.ops.segment_sum(jnp.ones((n,), jnp.float32), batch, num_segments=B)
    graph_vec = jax.ops.segment_sum(h2e, batch, num_segments=B) / jnp.maximum(counts, 1.0)[:, None]
    hidden = jax.nn.relu(graph_vec @ fw1 + fb1)
    return hidden @ fw2 + fb2


# speedup vs baseline: 21.0351x; 17.9125x over previous
import functools

import jax
import jax.numpy as jnp
from jax import lax
from jax.experimental import pallas as pl
from jax.experimental.pallas import tpu as pltpu
from jax.experimental.pallas import tpu_sc as plsc

_NBLK = 2000
_NP = 101376
_STAGE = 2048
_CH = 64


def _dense1_body(x_ref, w_ref, asrc_ref, adst_ref, h_ref, as_ref, ad_ref):
    h = jnp.dot(x_ref[...], w_ref[...], preferred_element_type=jnp.float32)
    h_ref[...] = h
    as_ref[...] = jnp.dot(h, asrc_ref[...], preferred_element_type=jnp.float32)
    ad_ref[...] = jnp.dot(h, adst_ref[...], preferred_element_type=jnp.float32)


def _dense1(x, W1, Asrc, Adst, heads):
    n = x.shape[0]
    d_out = W1.shape[1]
    grid = n // _NBLK
    return pl.pallas_call(
        _dense1_body,
        grid=(grid,),
        in_specs=[
            pl.BlockSpec((_NBLK, x.shape[1]), lambda i: (i, 0)),
            pl.BlockSpec((W1.shape[0], d_out), lambda i: (0, 0)),
            pl.BlockSpec((d_out, heads), lambda i: (0, 0)),
            pl.BlockSpec((d_out, heads), lambda i: (0, 0)),
        ],
        out_specs=[
            pl.BlockSpec((_NBLK, d_out), lambda i: (i, 0)),
            pl.BlockSpec((_NBLK, heads), lambda i: (i, 0)),
            pl.BlockSpec((_NBLK, heads), lambda i: (i, 0)),
        ],
        out_shape=[
            jax.ShapeDtypeStruct((n, d_out), jnp.float32),
            jax.ShapeDtypeStruct((n, heads), jnp.float32),
            jax.ShapeDtypeStruct((n, heads), jnp.float32),
        ],
    )(x, W1, Asrc, Adst)


def _block_diag_a(a):
    heads, c = a.shape
    eye = jnp.eye(heads, dtype=a.dtype)
    return (a[:, :, None] * eye[:, None, :]).reshape(heads * c, heads)


def _sc_gat(src2, dst2, asd, htab, *, C, H, K):
    R = _NP // K
    K2 = K // 2
    SH = (H - 1).bit_length()
    WROW = C + 16
    STRIPE = R // 16
    NFB = STRIPE // 8
    E2P = src2.shape[0]
    SLICE = E2P // 16
    NSTAGE = SLICE // _STAGE
    NV = C // 16
    NW = (_CH * H) // 16
    NG = _CH // 16

    mesh = plsc.VectorSubcoreMesh(core_axis_name="c", subcore_axis_name="s")

    def body(src_ref, dst_ref, asd_ref, h_ref, out_ref,
             accum, stage_s, stage_d, sel_s, sel_d, gsrc, gdst, lidx,
             asd_s, asd_d, hrows, wbuf, outb, fbuf, fout, apref,
             sem0, sem1, sem2):
        cid = lax.axis_index("c")
        sid = lax.axis_index("s")
        iota = lax.iota(jnp.int32, 16)
        zeros16 = jnp.zeros((16,), jnp.float32)

        def process_batch(valid, lo):
            for i in range(NG):
                rid = iota + (16 * i)
                ok = rid < valid
                sv = sel_s[pl.ds(16 * i, 16)]
                dv = sel_d[pl.ds(16 * i, 16)]
                gsrc[pl.ds(16 * i, 16)] = jnp.where(ok, sv, 0)
                gdst[pl.ds(16 * i, 16)] = jnp.where(ok, dv, 0)
                lidx[pl.ds(16 * i, 16)] = jnp.where(ok, dv - lo, 0)
            c0 = pltpu.async_copy(asd_ref.at[gsrc], asd_s, sem0)
            c1 = pltpu.async_copy(asd_ref.at[gdst], asd_d, sem1)
            c2 = pltpu.async_copy(h_ref.at[gsrc], hrows, sem2)
            c0.wait()
            c1.wait()
            c2.wait()
            for i in range(NW):
                p = iota + (16 * i)
                e = p >> SH
                hh = p & (H - 1)
                a = (plsc.load_gather(asd_s, [e, hh])
                     + plsc.load_gather(asd_d, [e, hh + 8]))
                a = jnp.where(a >= 0.0, a, 0.2 * a)
                w = jnp.where(e < valid, jnp.exp(a), 0.0)
                wbuf[pl.ds(16 * i, 16)] = w

            @pl.loop(0, _CH)
            def _compose(r):
                base = r * H
                for v in range(NV):
                    hv = hrows[r, pl.ds(16 * v, 16)]
                    mult = plsc.load_gather(wbuf, [base + ((iota + 16 * v) >> 6)])
                    outb[r, pl.ds(16 * v, 16)] = hv * mult
                tl = plsc.load_gather(wbuf, [base + (iota & (H - 1))])
                outb[r, pl.ds(C, 16)] = jnp.where(iota < H, tl, 0.0)

            pltpu.sync_copy(outb, accum.at[lidx], add=True)

        @pl.loop(0, K2)
        def _pass(j):
            k = cid * K2 + j
            lo = k * R
            hi = lo + R
            stripe0 = sid * STRIPE
            for r in range(8):
                for v in range(WROW // 16):
                    outb[r, pl.ds(16 * v, 16)] = zeros16

            @pl.loop(0, NFB)
            def _zero(b):
                zr = pl.multiple_of(stripe0 + 8 * b, 8)
                pltpu.sync_copy(outb.at[pl.ds(0, 8), :],
                                accum.at[pl.ds(zr, 8), :])

            plsc.subcore_barrier()
            apref[0] = 0

            @pl.loop(0, NSTAGE)
            def _stage(st):
                sb = pl.multiple_of(sid * SLICE + st * _STAGE, 8)
                pltpu.sync_copy(src_ref.at[pl.ds(sb, _STAGE)], stage_s)
                pltpu.sync_copy(dst_ref.at[pl.ds(sb, _STAGE)], stage_d)

                @pl.loop(0, _STAGE // _CH)
                def _chunk(cc):
                    off = cc * _CH
                    for g in range(NG):
                        ap = apref[0]
                        s_v = stage_s[pl.ds(off + 16 * g, 16)]
                        d_v = stage_d[pl.ds(off + 16 * g, 16)]
                        m = (d_v >= lo) & (d_v < hi)
                        mi = m.astype(jnp.int32)
                        inc = plsc.cumsum(mi)
                        idx = jnp.where(m, ap + inc - mi, 2 * _CH + 16)
                        plsc.store_scatter(sel_s, [idx], s_v)
                        plsc.store_scatter(sel_d, [idx], d_v)
                        apref[0] = ap + jnp.sum(mi)
                    ap = apref[0]

                    @pl.when(ap >= _CH)
                    def _drain():
                        process_batch(jnp.int32(_CH), lo)
                        for i in range(NG):
                            sel_s[pl.ds(16 * i, 16)] = sel_s[pl.ds(_CH + 16 * i, 16)]
                            sel_d[pl.ds(16 * i, 16)] = sel_d[pl.ds(_CH + 16 * i, 16)]
                        apref[0] = ap - _CH

            apt = apref[0]

            @pl.when(apt > 0)
            def _tail():
                process_batch(apt, lo)

            plsc.subcore_barrier()

            @pl.loop(0, NFB)
            def _flush(b):
                row0 = pl.multiple_of(stripe0 + 8 * b, 8)
                pltpu.sync_copy(accum.at[pl.ds(row0, 8), :], fbuf)
                for r in range(8):
                    rr = jnp.full((16,), r, jnp.int32)
                    for v in range(NV):
                        num = fbuf[r, pl.ds(16 * v, 16)]
                        den = plsc.load_gather(
                            fbuf, [rr, C + ((iota + 16 * v) >> 6)])
                        fout[r, pl.ds(16 * v, 16)] = num / (den + 1e-16)
                orow = pl.multiple_of(lo + row0, 8)
                pltpu.sync_copy(fout, out_ref.at[pl.ds(orow, 8), :])

    fn = pl.kernel(
        body,
        out_type=jax.ShapeDtypeStruct((_NP, C), jnp.float32),
        mesh=mesh,
        compiler_params=pltpu.CompilerParams(needs_layout_passes=False,
                                             use_tc_tiling_on_sc=False),
        scratch_types=[
            pltpu.VMEM_SHARED((R, WROW), jnp.float32),
            pltpu.VMEM((_STAGE,), jnp.int32),
            pltpu.VMEM((_STAGE,), jnp.int32),
            pltpu.VMEM((2 * _CH + 32,), jnp.int32),
            pltpu.VMEM((2 * _CH + 32,), jnp.int32),
            pltpu.VMEM((_CH,), jnp.int32),
            pltpu.VMEM((_CH,), jnp.int32),
            pltpu.VMEM((_CH,), jnp.int32),
            pltpu.VMEM((_CH, 16), jnp.float32),
            pltpu.VMEM((_CH, 16), jnp.float32),
            pltpu.VMEM((_CH, C), jnp.float32),
            pltpu.VMEM((_CH * H,), jnp.float32),
            pltpu.VMEM((_CH, WROW), jnp.float32),
            pltpu.VMEM((8, WROW), jnp.float32),
            pltpu.VMEM((8, C), jnp.float32),
            pltpu.SMEM((1,), jnp.int32),
            pltpu.SemaphoreType.DMA,
            pltpu.SemaphoreType.DMA,
            pltpu.SemaphoreType.DMA,
        ],
    )
    return fn(src2, dst2, asd, htab)


def _pack_asd(alpha_s, alpha_d):
    n, heads = alpha_s.shape
    asd = jnp.zeros((n, 16), jnp.float32)
    asd = asd.at[:, 0:heads].set(alpha_s)
    asd = asd.at[:, 8:8 + heads].set(alpha_d)
    return asd


def _batch_norm(x, gamma, beta, eps=1e-5):
    mu = jnp.mean(x, axis=0)
    var = jnp.var(x, axis=0)
    return (x - mu) / jnp.sqrt(var + eps) * gamma + beta


def kernel(x, edge_index, batch, W1, a_src1, a_dst1, b1, g1, be1,
           W2, a_src2, a_dst2, b2, g2, be2, fw1, fb1, fw2, fb2):
    n = x.shape[0]
    B = 128
    loops = jnp.arange(n, dtype=edge_index.dtype)
    e2 = edge_index.shape[1] + n
    e2p = ((e2 + 16 * _STAGE - 1) // (16 * _STAGE)) * (16 * _STAGE)
    pad = e2p - e2
    src2 = jnp.concatenate([edge_index[0], loops,
                            jnp.zeros((pad,), jnp.int32)])
    dst2 = jnp.concatenate([edge_index[1], loops,
                            jnp.full((pad,), -1, jnp.int32)])

    h1, as1, ad1 = _dense1(x, W1, _block_diag_a(a_src1), _block_diag_a(a_dst1), 4)
    o1 = _sc_gat(src2, dst2, _pack_asd(as1, ad1), h1, C=256, H=4, K=24)[:n] + b1
    h1e = jax.nn.elu(_batch_norm(o1, g1, be1))

    h2, as2, ad2 = _dense1(h1e, W2, _block_diag_a(a_src2), _block_diag_a(a_dst2), 1)
    o2 = _sc_gat(src2, dst2, _pack_asd(as2, ad2), h2, C=64, H=1, K=8)[:n] + b2
    h2e = jax.nn.elu(_batch_norm(o2, g2, be2))

    counts = jax.ops.segment_sum(jnp.ones((n,), jnp.float32), batch, num_segments=B)
    graph_vec = jax.ops.segment_sum(h2e, batch, num_segments=B) / jnp.maximum(counts, 1.0)[:, None]
    hidden = jax.nn.relu(graph_vec @ fw1 + fb1)
    return hidden @ fw2 + fb2


# speedup vs baseline: 22.4559x; 1.0675x over previous
import functools

import jax
import jax.numpy as jnp
from jax import lax
from jax.experimental import pallas as pl
from jax.experimental.pallas import tpu as pltpu
from jax.experimental.pallas import tpu_sc as plsc

_NBLK = 2000
_NP = 101376
_STAGE = 2048


def _dense1_body(x_ref, w_ref, asrc_ref, adst_ref, h_ref, as_ref, ad_ref):
    h = jnp.dot(x_ref[...], w_ref[...], preferred_element_type=jnp.float32)
    h_ref[...] = h
    as_ref[...] = jnp.dot(h, asrc_ref[...], preferred_element_type=jnp.float32)
    ad_ref[...] = jnp.dot(h, adst_ref[...], preferred_element_type=jnp.float32)


def _dense1(x, W1, Asrc, Adst, heads):
    n = x.shape[0]
    d_out = W1.shape[1]
    grid = n // _NBLK
    return pl.pallas_call(
        _dense1_body,
        grid=(grid,),
        in_specs=[
            pl.BlockSpec((_NBLK, x.shape[1]), lambda i: (i, 0)),
            pl.BlockSpec((W1.shape[0], d_out), lambda i: (0, 0)),
            pl.BlockSpec((d_out, heads), lambda i: (0, 0)),
            pl.BlockSpec((d_out, heads), lambda i: (0, 0)),
        ],
        out_specs=[
            pl.BlockSpec((_NBLK, d_out), lambda i: (i, 0)),
            pl.BlockSpec((_NBLK, heads), lambda i: (i, 0)),
            pl.BlockSpec((_NBLK, heads), lambda i: (i, 0)),
        ],
        out_shape=[
            jax.ShapeDtypeStruct((n, d_out), jnp.float32),
            jax.ShapeDtypeStruct((n, heads), jnp.float32),
            jax.ShapeDtypeStruct((n, heads), jnp.float32),
        ],
    )(x, W1, Asrc, Adst)


def _block_diag_a(a):
    heads, c = a.shape
    eye = jnp.eye(heads, dtype=a.dtype)
    return (a[:, :, None] * eye[:, None, :]).reshape(heads * c, heads)


def _sc_gat(src2, dst2, asd, htab, *, C, H, K, CH):
    R = _NP // K
    K2 = K // 2
    SH = (H - 1).bit_length()
    WROW = C + 16
    STRIPE = R // 16
    NFB = STRIPE // 8
    E2P = src2.shape[0]
    SLICE = E2P // 16
    NSTAGE = SLICE // _STAGE
    NV = C // 16
    NW = (CH * H) // 16
    NG = CH // 16

    mesh = plsc.VectorSubcoreMesh(core_axis_name="c", subcore_axis_name="s")

    def body(src_ref, dst_ref, asd_ref, h_ref, out_ref,
             accum, stage_s, stage_d, sel_s, sel_d, gsrc, gdst, lidx,
             asd_s, asd_d, hrows, wbuf, outb, fbuf, fout, apref,
             sem0, sem1, sem2):
        cid = lax.axis_index("c")
        sid = lax.axis_index("s")
        iota = lax.iota(jnp.int32, 16)
        zeros16 = jnp.zeros((16,), jnp.float32)

        def process_batch(valid, lo):
            for i in range(NG):
                rid = iota + (16 * i)
                ok = rid < valid
                sv = sel_s[pl.ds(16 * i, 16)]
                dv = sel_d[pl.ds(16 * i, 16)]
                gsrc[pl.ds(16 * i, 16)] = jnp.where(ok, sv, 0)
                gdst[pl.ds(16 * i, 16)] = jnp.where(ok, dv, 0)
                lidx[pl.ds(16 * i, 16)] = jnp.where(ok, dv - lo, 0)
            c0 = pltpu.async_copy(asd_ref.at[gsrc], asd_s, sem0)
            c1 = pltpu.async_copy(asd_ref.at[gdst], asd_d, sem1)
            c2 = pltpu.async_copy(h_ref.at[gsrc], hrows, sem2)
            c0.wait()
            c1.wait()
            c2.wait()
            for i in range(NW):
                p = iota + (16 * i)
                e = p >> SH
                hh = p & (H - 1)
                a = (plsc.load_gather(asd_s, [e, hh])
                     + plsc.load_gather(asd_d, [e, hh + 8]))
                a = jnp.where(a >= 0.0, a, 0.2 * a)
                w = jnp.where(e < valid, jnp.exp(a), 0.0)
                wbuf[pl.ds(16 * i, 16)] = w

            @pl.loop(0, CH)
            def _compose(r):
                base = r * H
                vh = NV // H
                for hd in range(H):
                    mult = plsc.load_gather(
                        wbuf, [jnp.broadcast_to(base + hd, (16,))])
                    for v in range(hd * vh, (hd + 1) * vh):
                        hv = hrows[r, pl.ds(16 * v, 16)]
                        outb[r, pl.ds(16 * v, 16)] = hv * mult
                tl = plsc.load_gather(wbuf, [base + (iota & (H - 1))])
                outb[r, pl.ds(C, 16)] = jnp.where(iota < H, tl, 0.0)

            pltpu.sync_copy(outb, accum.at[lidx], add=True)

        @pl.loop(0, K2)
        def _pass(j):
            k = cid * K2 + j
            lo = k * R
            hi = lo + R
            stripe0 = sid * STRIPE
            for r in range(8):
                for v in range(WROW // 16):
                    outb[r, pl.ds(16 * v, 16)] = zeros16

            @pl.loop(0, NFB)
            def _zero(b):
                zr = pl.multiple_of(stripe0 + 8 * b, 8)
                pltpu.sync_copy(outb.at[pl.ds(0, 8), :],
                                accum.at[pl.ds(zr, 8), :])

            plsc.subcore_barrier()
            apref[0] = 0

            @pl.loop(0, NSTAGE)
            def _stage(st):
                sb = pl.multiple_of(sid * SLICE + st * _STAGE, 8)
                pltpu.sync_copy(src_ref.at[pl.ds(sb, _STAGE)], stage_s)
                pltpu.sync_copy(dst_ref.at[pl.ds(sb, _STAGE)], stage_d)

                @pl.loop(0, _STAGE // CH)
                def _chunk(cc):
                    off = cc * CH
                    for g in range(NG):
                        ap = apref[0]
                        s_v = stage_s[pl.ds(off + 16 * g, 16)]
                        d_v = stage_d[pl.ds(off + 16 * g, 16)]
                        m = (d_v >= lo) & (d_v < hi)
                        mi = m.astype(jnp.int32)
                        inc = plsc.cumsum(mi)
                        idx = jnp.where(m, ap + inc - mi, 2 * CH + 16)
                        plsc.store_scatter(sel_s, [idx], s_v)
                        plsc.store_scatter(sel_d, [idx], d_v)
                        apref[0] = ap + jnp.sum(mi)
                    ap = apref[0]

                    @pl.when(ap >= CH)
                    def _drain():
                        process_batch(jnp.int32(CH), lo)
                        for i in range(NG):
                            sel_s[pl.ds(16 * i, 16)] = sel_s[pl.ds(CH + 16 * i, 16)]
                            sel_d[pl.ds(16 * i, 16)] = sel_d[pl.ds(CH + 16 * i, 16)]
                        apref[0] = ap - CH

            apt = apref[0]

            @pl.when(apt > 0)
            def _tail():
                process_batch(apt, lo)

            plsc.subcore_barrier()

            @pl.loop(0, NFB)
            def _flush(b):
                row0 = pl.multiple_of(stripe0 + 8 * b, 8)
                pltpu.sync_copy(accum.at[pl.ds(row0, 8), :], fbuf)
                for r in range(8):
                    rr = jnp.full((16,), r, jnp.int32)
                    for v in range(NV):
                        num = fbuf[r, pl.ds(16 * v, 16)]
                        den = plsc.load_gather(
                            fbuf, [rr, C + ((iota + 16 * v) >> 6)])
                        fout[r, pl.ds(16 * v, 16)] = num / (den + 1e-16)
                orow = pl.multiple_of(lo + row0, 8)
                pltpu.sync_copy(fout, out_ref.at[pl.ds(orow, 8), :])

    fn = pl.kernel(
        body,
        out_type=jax.ShapeDtypeStruct((_NP, C), jnp.float32),
        mesh=mesh,
        compiler_params=pltpu.CompilerParams(needs_layout_passes=False,
                                             use_tc_tiling_on_sc=False),
        scratch_types=[
            pltpu.VMEM_SHARED((R, WROW), jnp.float32),
            pltpu.VMEM((_STAGE,), jnp.int32),
            pltpu.VMEM((_STAGE,), jnp.int32),
            pltpu.VMEM((2 * CH + 32,), jnp.int32),
            pltpu.VMEM((2 * CH + 32,), jnp.int32),
            pltpu.VMEM((CH,), jnp.int32),
            pltpu.VMEM((CH,), jnp.int32),
            pltpu.VMEM((CH,), jnp.int32),
            pltpu.VMEM((CH, 16), jnp.float32),
            pltpu.VMEM((CH, 16), jnp.float32),
            pltpu.VMEM((CH, C), jnp.float32),
            pltpu.VMEM((CH * H,), jnp.float32),
            pltpu.VMEM((CH, WROW), jnp.float32),
            pltpu.VMEM((8, WROW), jnp.float32),
            pltpu.VMEM((8, C), jnp.float32),
            pltpu.SMEM((1,), jnp.int32),
            pltpu.SemaphoreType.DMA,
            pltpu.SemaphoreType.DMA,
            pltpu.SemaphoreType.DMA,
        ],
    )
    return fn(src2, dst2, asd, htab)


def _pack_asd(alpha_s, alpha_d):
    n, heads = alpha_s.shape
    asd = jnp.zeros((n, 16), jnp.float32)
    asd = asd.at[:, 0:heads].set(alpha_s)
    asd = asd.at[:, 8:8 + heads].set(alpha_d)
    return asd


def _batch_norm(x, gamma, beta, eps=1e-5):
    mu = jnp.mean(x, axis=0)
    var = jnp.var(x, axis=0)
    return (x - mu) / jnp.sqrt(var + eps) * gamma + beta


def kernel(x, edge_index, batch, W1, a_src1, a_dst1, b1, g1, be1,
           W2, a_src2, a_dst2, b2, g2, be2, fw1, fb1, fw2, fb2):
    n = x.shape[0]
    B = 128
    loops = jnp.arange(n, dtype=edge_index.dtype)
    e2 = edge_index.shape[1] + n
    e2p = ((e2 + 16 * _STAGE - 1) // (16 * _STAGE)) * (16 * _STAGE)
    pad = e2p - e2
    src2 = jnp.concatenate([edge_index[0], loops,
                            jnp.zeros((pad,), jnp.int32)])
    dst2 = jnp.concatenate([edge_index[1], loops,
                            jnp.full((pad,), -1, jnp.int32)])

    h1, as1, ad1 = _dense1(x, W1, _block_diag_a(a_src1), _block_diag_a(a_dst1), 4)
    o1 = _sc_gat(src2, dst2, _pack_asd(as1, ad1), h1, C=256, H=4, K=24, CH=64)[:n] + b1
    h1e = jax.nn.elu(_batch_norm(o1, g1, be1))

    h2, as2, ad2 = _dense1(h1e, W2, _block_diag_a(a_src2), _block_diag_a(a_dst2), 1)
    o2 = _sc_gat(src2, dst2, _pack_asd(as2, ad2), h2, C=64, H=1, K=8, CH=128)[:n] + b2
    h2e = jax.nn.elu(_batch_norm(o2, g2, be2))

    counts = jax.ops.segment_sum(jnp.ones((n,), jnp.float32), batch, num_segments=B)
    graph_vec = jax.ops.segment_sum(h2e, batch, num_segments=B) / jnp.maximum(counts, 1.0)[:, None]
    hidden = jax.nn.relu(graph_vec @ fw1 + fb1)
    return hidden @ fw2 + fb2
